# Initial kernel scaffold; baseline (speedup 1.0000x reference)
import jax
import jax.numpy as jnp
from jax.experimental import pallas as pl


def kernel(cart, neighlist, shifts, species, W_emb, b_emb, W_rad, b_rad, W_msg, b_msg, W_out, b_out):
    raise NotImplementedError("write your pallas kernel here")


# speedup vs baseline: 77.6053x; 77.6053x over previous
import functools

import jax
import jax.numpy as jnp
import numpy as np
from jax import lax
from jax.experimental import pallas as pl
from jax.experimental.pallas import tpu as pltpu
from jax.experimental.pallas import tpu_sc as plsc

MAX_L = 2
NWAVE = 8
CUTOFF = 5.0
ITER_LOOP = 3
N_NODES = 50000
N_EDGES = 800000
NSPEC = 8
NORB = NWAVE * (MAX_L + 1)
NCOEF = 2 * (NWAVE + 1)

NP = 50176
EP = 802816
EPP = EP + 128
NC, NS, NW = 2, 16, 32
C = 128
ET = EP // NW
NCHUNK = ET // C
TROWS = NP // NS
ZB = 98
NZCP = TROWS // ZB

BN = 1024
GN = NP // BN
BE = 4096
GE = EP // BE

_f32 = jnp.float32


def _silu(x):
    return x * jax.nn.sigmoid(x)


def _full16(v):
    return jnp.full((16,), v, jnp.int32)


def _sc_setup_body(cart_hbm, idxn_hbm, idxc_hbm, sh_hbm, dv_hbm,
                   idxn_v, idxc_v, sh_v, gn_v, gc_v, dv_v, sem1, sem2):
    cid = lax.axis_index("c")
    sid = lax.axis_index("s")
    base = (sid * NC + cid) * ET

    def chunk(i, carry):
        e0 = base + i * C
        pltpu.sync_copy(idxn_hbm.at[pl.ds(e0, C)], idxn_v)
        pltpu.sync_copy(idxc_hbm.at[pl.ds(e0, C)], idxc_v)
        pltpu.sync_copy(sh_hbm.at[:, pl.ds(e0, C)], sh_v)
        cp1 = pltpu.async_copy(cart_hbm.at[idxn_v], gn_v, sem1)
        cp2 = pltpu.async_copy(cart_hbm.at[idxc_v], gc_v, sem2)
        cp1.wait()
        cp2.wait()
        for g in range(C // 16):
            rid = lax.iota(jnp.int32, 16) + g * 16
            r2 = jnp.zeros((16,), _f32)
            for j in range(3):
                xn = plsc.load_gather(gn_v, [rid, _full16(j)])
                xc = plsc.load_gather(gc_v, [rid, _full16(j)])
                dj = xn - xc + sh_v[j, pl.ds(g * 16, 16)]
                dv_v[j, pl.ds(g * 16, 16)] = dj
                r2 = r2 + dj * dj
            dv_v[3, pl.ds(g * 16, 16)] = r2
        pltpu.sync_copy(dv_v, dv_hbm.at[:, pl.ds(e0, C)])
        return carry

    lax.fori_loop(0, NCHUNK, chunk, 0)


def _sc_setup(cartp, idxn, idxc, shifts_t):
    mesh = plsc.VectorSubcoreMesh(core_axis_name="c", subcore_axis_name="s")
    f = pl.kernel(
        _sc_setup_body,
        out_type=jax.ShapeDtypeStruct((4, EP), _f32),
        mesh=mesh,
        compiler_params=pltpu.CompilerParams(needs_layout_passes=False, use_tc_tiling_on_sc=False),
        scratch_types=[
            pltpu.VMEM((C,), jnp.int32),
            pltpu.VMEM((C,), jnp.int32),
            pltpu.VMEM((4, C), _f32),
            pltpu.VMEM((C, 8), _f32),
            pltpu.VMEM((C, 8), _f32),
            pltpu.VMEM((4, C), _f32),
            pltpu.SemaphoreType.DMA,
            pltpu.SemaphoreType.DMA,
        ],
    )
    return f(cartp, idxn, idxc, shifts_t)


def _sc_edge_body(with_mpc, p_hbm, ec_hbm, idxn_hbm, idxc_hbm, out_hbm,
                  idxn_v0, idxn_v1, idxc_v0, idxc_v1, ec_v0, ec_v1,
                  rows_v0, rows_v1, out_v0, out_v1, zb_v, acc_sh,
                  asem0, asem1, gsem0, gsem1, ssem0, ssem1):
    cid = lax.axis_index("c")
    sid = lax.axis_index("s")
    idxn_v = (idxn_v0, idxn_v1)
    idxc_v = (idxc_v0, idxc_v1)
    ec_v = (ec_v0, ec_v1)
    rows_v = (rows_v0, rows_v1)
    out_v = (out_v0, out_v1)
    asem = (asem0, asem1)
    gsem = (gsem0, gsem1)
    ssem = (ssem0, ssem1)

    def zrow(r, carry):
        zb_v[r, pl.ds(0, 16)] = jnp.zeros((16,), _f32)
        zb_v[r, pl.ds(16, 16)] = jnp.zeros((16,), _f32)
        return carry

    lax.fori_loop(0, ZB, zrow, 0)

    def zcp(i, carry):
        pltpu.sync_copy(zb_v, acc_sh.at[pl.ds(sid * TROWS + i * ZB, ZB)])
        return carry

    lax.fori_loop(0, NZCP, zcp, 0)
    plsc.subcore_barrier()

    base = (sid * NC + cid) * ET

    def issue_a(c, nb):
        e1 = base + c * C
        a1 = pltpu.async_copy(idxn_hbm.at[pl.ds(e1, C)], idxn_v[nb], asem[nb])
        a2 = pltpu.async_copy(idxc_hbm.at[pl.ds(e1, C)], idxc_v[nb], asem[nb])
        a3 = pltpu.async_copy(ec_hbm.at[:, pl.ds(e1, C)], ec_v[nb], asem[nb])
        return a1, a2, a3

    def compute(b):
        def group(g, carry):
            rid = lax.iota(jnp.int32, 16) + g * 16
            cut = plsc.load_gather(ec_v[b], [_full16(0), rid])
            cx = plsc.load_gather(ec_v[b], [_full16(1), rid])
            cy = plsc.load_gather(ec_v[b], [_full16(2), rid])
            cz = plsc.load_gather(ec_v[b], [_full16(3), rid])
            for k in range(NWAVE):
                nck = plsc.load_gather(rows_v[b], [rid, _full16(k)])
                plsc.store_scatter(out_v[b], [rid, _full16(k)], cut * nck)
            for k in range(NWAVE):
                nc2 = plsc.load_gather(rows_v[b], [rid, _full16(8 + k)])
                for j, cj in enumerate((cx, cy, cz)):
                    if with_mpc:
                        mpcv = plsc.load_gather(rows_v[b],
                                                [rid, _full16(16 + j * 8 + k)])
                        val = cj * nc2 + mpcv
                    else:
                        val = cj * nc2
                    plsc.store_scatter(out_v[b], [rid, _full16(8 + j * 8 + k)],
                                       val)
            return carry

        lax.fori_loop(0, C // 16, group, 0)

    def wait_scatter(b):
        pltpu.make_async_copy(out_v[b], acc_sh.at[idxc_v[b]], ssem[b]).wait()

    def wait_gather(b):
        pltpu.make_async_copy(p_hbm.at[idxn_v[b]], rows_v[b], gsem[b]).wait()

    def step(c, b, nb, first):
        if first:
            @pl.when(c >= 1)
            def _w():
                wait_scatter(nb)
        else:
            wait_scatter(nb)
        a1, a2, a3 = issue_a(c + 1, nb)
        wait_gather(b)
        compute(b)
        pltpu.async_copy(out_v[b], acc_sh.at[idxc_v[b]], ssem[b], add=True)
        a1.wait()
        a2.wait()
        a3.wait()
        pltpu.async_copy(p_hbm.at[idxn_v[nb]], rows_v[nb], gsem[nb])

    pltpu.sync_copy(idxn_hbm.at[pl.ds(base, C)], idxn_v[0])
    pltpu.sync_copy(idxc_hbm.at[pl.ds(base, C)], idxc_v[0])
    pltpu.sync_copy(ec_hbm.at[:, pl.ds(base, C)], ec_v[0])
    pltpu.async_copy(p_hbm.at[idxn_v[0]], rows_v[0], gsem[0])

    def pair(i2, carry):
        c0 = i2 * 2
        step(c0, 0, 1, True)
        step(c0 + 1, 1, 0, False)
        return carry

    lax.fori_loop(0, NCHUNK // 2, pair, 0)
    wait_scatter(1)
    wait_gather(0)
    plsc.subcore_barrier()
    r0 = sid * TROWS
    pltpu.sync_copy(acc_sh.at[pl.ds(r0, TROWS)],
                    out_hbm.at[cid, pl.ds(r0, TROWS)])


def _sc_edge_call(p, ect, idxn, idxc, pw, with_mpc):
    mesh = plsc.VectorSubcoreMesh(core_axis_name="c", subcore_axis_name="s")
    f = pl.kernel(
        functools.partial(_sc_edge_body, with_mpc),
        out_type=jax.ShapeDtypeStruct((NC, NP, 32), _f32),
        mesh=mesh,
        compiler_params=pltpu.CompilerParams(needs_layout_passes=False, use_tc_tiling_on_sc=False),
        scratch_types=[
            pltpu.VMEM((C,), jnp.int32),
            pltpu.VMEM((C,), jnp.int32),
            pltpu.VMEM((C,), jnp.int32),
            pltpu.VMEM((C,), jnp.int32),
            pltpu.VMEM((4, C), _f32),
            pltpu.VMEM((4, C), _f32),
            pltpu.VMEM((C, pw), _f32),
            pltpu.VMEM((C, pw), _f32),
            pltpu.VMEM((C, 32), _f32),
            pltpu.VMEM((C, 32), _f32),
            pltpu.VMEM((ZB, 32), _f32),
            pltpu.VMEM_SHARED((NP, 32), _f32),
            pltpu.SemaphoreType.DMA,
            pltpu.SemaphoreType.DMA,
            pltpu.SemaphoreType.DMA,
            pltpu.SemaphoreType.DMA,
            pltpu.SemaphoreType.DMA,
            pltpu.SemaphoreType.DMA,
        ],
    )
    return f(p, ect, idxn, idxc)


def _sc_edge(p, ect, idxn, idxc):
    return _sc_edge_call(p, ect, idxn, idxc, 40, True)


def _sc_edge0(p0, ect, idxn, idxc):
    return _sc_edge_call(p0, ect, idxn, idxc, 16, False)


def _tc_prep_body(dv_ref, ec_ref):
    pid = pl.program_id(0)
    dv = dv_ref[...]
    dx = dv[0:1, :]
    dy = dv[1:2, :]
    dz = dv[2:3, :]
    r2 = dv[3:4, :]
    d = jnp.sqrt(r2)
    cut = jnp.power(0.5 * jnp.cos(d * (np.pi / CUTOFF)) + 0.5, 3)
    col = lax.broadcasted_iota(jnp.int32, (1, BE), 1) + pid * BE
    cut = jnp.where(col < N_EDGES, cut, 0.0)
    ec_ref[...] = jnp.concatenate([cut, cut * dx, cut * dy, cut * dz], axis=0)


def _tc_prep(dvt):
    return pl.pallas_call(
        _tc_prep_body,
        grid=(GE,),
        in_specs=[pl.BlockSpec((4, BE), lambda i: (0, i))],
        out_specs=pl.BlockSpec((4, BE), lambda i: (0, i)),
        out_shape=jax.ShapeDtypeStruct((4, EP), _f32),
    )(dvt)


def _tc_emb_body(sp_ref, we_ref, be_ref, icf_ref, p_ref):
    pid = pl.program_id(0)
    icf = _silu(jnp.dot(sp_ref[...], we_ref[...],
                        preferred_element_type=_f32) + be_ref[...])
    icf_ref[...] = icf
    rows = lax.broadcasted_iota(jnp.int32, (BN, 1), 0) + pid * BN
    mask = (rows < N_NODES).astype(_f32)
    p = jnp.concatenate([icf[:, 0:8], icf[:, 9:17]], axis=1)
    p_ref[...] = p * mask


def _tc_emb(speciesp, w_emb, b_emb):
    return pl.pallas_call(
        _tc_emb_body,
        grid=(GN,),
        in_specs=[
            pl.BlockSpec((BN, NSPEC), lambda i: (i, 0)),
            pl.BlockSpec((NSPEC, NCOEF), lambda i: (0, 0)),
            pl.BlockSpec((1, NCOEF), lambda i: (0, 0)),
        ],
        out_specs=[
            pl.BlockSpec((BN, NCOEF), lambda i: (i, 0)),
            pl.BlockSpec((BN, 16), lambda i: (i, 0)),
        ],
        out_shape=[
            jax.ShapeDtypeStruct((NP, NCOEF), _f32),
            jax.ShapeDtypeStruct((NP, 16), _f32),
        ],
    )(speciesp, w_emb, b_emb)


def _tc_node_body(acc0_ref, acc1_ref, icf_ref, dens_ref, mpc_ref,
                  wr_ref, br_ref, wm_ref, bm_ref,
                  dens_out, mpc_out, icf_out, p_out):
    pid = pl.program_id(0)
    acc = acc0_ref[0] + acc1_ref[0]
    mpd = acc[:, 0:8]
    ss = acc[:, 8:32]
    icf = icf_ref[...]
    mpc = mpc_ref[...] + ss
    radial = _silu(jnp.dot(mpd * icf[:, 8:9], wr_ref[...],
                           preferred_element_type=_f32) + br_ref[...])
    c = icf[:, 17:18]
    x = mpc[:, 0:8] * c
    y = mpc[:, 8:16] * c
    z = mpc[:, 16:24] * c
    r2 = x * x + y * y + z * z
    ang2 = ((x * y) ** 2 + (y * z) ** 2 + (3.0 * z * z - r2) ** 2
            + (x * z) ** 2 + (x * x - y * y) ** 2)
    dens = dens_ref[...] + jnp.concatenate(
        [radial, radial * r2, radial * ang2], axis=1)
    dens_out[...] = dens
    mpc_out[...] = mpc
    icf_new = _silu(jnp.dot(dens, wm_ref[...],
                            preferred_element_type=_f32) + bm_ref[...])
    icf_out[...] = icf_new
    rows = lax.broadcasted_iota(jnp.int32, (BN, 1), 0) + pid * BN
    mask = (rows < N_NODES).astype(_f32)
    p = jnp.concatenate([icf_new[:, 0:8], icf_new[:, 9:17], mpc], axis=1)
    p_out[...] = p * mask


def _tc_node(acc, icf, dens, mpc, w_rad, b_rad, w_msg_i, b_msg_i):
    return pl.pallas_call(
        _tc_node_body,
        grid=(GN,),
        in_specs=[
            pl.BlockSpec((1, BN, 32), lambda i: (0, i, 0)),
            pl.BlockSpec((1, BN, 32), lambda i: (1, i, 0)),
            pl.BlockSpec((BN, NCOEF), lambda i: (i, 0)),
            pl.BlockSpec((BN, NORB), lambda i: (i, 0)),
            pl.BlockSpec((BN, NORB), lambda i: (i, 0)),
            pl.BlockSpec((NWAVE, NWAVE), lambda i: (0, 0)),
            pl.BlockSpec((1, NWAVE), lambda i: (0, 0)),
            pl.BlockSpec((NORB, NCOEF), lambda i: (0, 0)),
            pl.BlockSpec((1, NCOEF), lambda i: (0, 0)),
        ],
        out_specs=[
            pl.BlockSpec((BN, NORB), lambda i: (i, 0)),
            pl.BlockSpec((BN, NORB), lambda i: (i, 0)),
            pl.BlockSpec((BN, NCOEF), lambda i: (i, 0)),
            pl.BlockSpec((BN, 40), lambda i: (i, 0)),
        ],
        out_shape=[
            jax.ShapeDtypeStruct((NP, NORB), _f32),
            jax.ShapeDtypeStruct((NP, NORB), _f32),
            jax.ShapeDtypeStruct((NP, NCOEF), _f32),
            jax.ShapeDtypeStruct((NP, 40), _f32),
        ],
    )(acc, acc, icf, dens, mpc, w_rad, b_rad, w_msg_i, b_msg_i)


def _tc_node_last_body(acc0_ref, acc1_ref, icf_ref, dens_ref, mpc_ref,
                       wr_ref, br_ref, wo_ref, out_ref):
    pid = pl.program_id(0)
    acc = acc0_ref[0] + acc1_ref[0]
    mpd = acc[:, 0:8]
    ss = acc[:, 8:32]
    icf = icf_ref[...]
    mpc = mpc_ref[...] + ss
    radial = _silu(jnp.dot(mpd * icf[:, 8:9], wr_ref[...],
                           preferred_element_type=_f32) + br_ref[...])
    c = icf[:, 17:18]
    x = mpc[:, 0:8] * c
    y = mpc[:, 8:16] * c
    z = mpc[:, 16:24] * c
    r2 = x * x + y * y + z * z
    ang2 = ((x * y) ** 2 + (y * z) ** 2 + (3.0 * z * z - r2) ** 2
            + (x * z) ** 2 + (x * x - y * y) ** 2)
    dens = dens_ref[...] + jnp.concatenate(
        [radial, radial * r2, radial * ang2], axis=1)
    rows = lax.broadcasted_iota(jnp.int32, (BN, 1), 0) + pid * BN
    mask = (rows < N_NODES).astype(_f32)
    part = jnp.sum(jnp.dot(dens * mask, wo_ref[...],
                           preferred_element_type=_f32))

    @pl.when(pid == 0)
    def _init():
        out_ref[...] = jnp.zeros((1, 1), _f32)

    out_ref[...] = out_ref[...] + part


def _tc_node_last(acc, icf, dens, mpc, w_rad, b_rad, w_out):
    return pl.pallas_call(
        _tc_node_last_body,
        grid=(GN,),
        in_specs=[
            pl.BlockSpec((1, BN, 32), lambda i: (0, i, 0)),
            pl.BlockSpec((1, BN, 32), lambda i: (1, i, 0)),
            pl.BlockSpec((BN, NCOEF), lambda i: (i, 0)),
            pl.BlockSpec((BN, NORB), lambda i: (i, 0)),
            pl.BlockSpec((BN, NORB), lambda i: (i, 0)),
            pl.BlockSpec((NWAVE, NWAVE), lambda i: (0, 0)),
            pl.BlockSpec((1, NWAVE), lambda i: (0, 0)),
            pl.BlockSpec((NORB, 1), lambda i: (0, 0)),
        ],
        out_specs=pl.BlockSpec((1, 1), lambda i: (0, 0)),
        out_shape=jax.ShapeDtypeStruct((1, 1), _f32),
    )(acc, acc, icf, dens, mpc, w_rad, b_rad, w_out)


def kernel(cart, neighlist, shifts, species, W_emb, b_emb, W_rad, b_rad,
           W_msg, b_msg, W_out, b_out):
    idx_c = neighlist[0].astype(jnp.int32)
    idx_n = neighlist[1].astype(jnp.int32)
    idx_c = jnp.pad(idx_c, (0, EPP - N_EDGES), constant_values=N_NODES)
    idx_n = jnp.pad(idx_n, (0, EPP - N_EDGES), constant_values=N_NODES)
    shifts_t = jnp.pad(shifts.T.astype(_f32), ((0, 1), (0, EP - N_EDGES)))
    cartp = jnp.pad(cart.astype(_f32), ((0, NP - N_NODES), (0, 5)))
    speciesp = jnp.pad(species.astype(_f32), ((0, NP - N_NODES), (0, 0)))

    dvt = _sc_setup(cartp, idx_n, idx_c, shifts_t)
    ect = jnp.pad(_tc_prep(dvt), ((0, 0), (0, EPP - EP)))
    icf, p = _tc_emb(speciesp, W_emb, b_emb.reshape(1, NCOEF))

    dens = jnp.zeros((NP, NORB), _f32)
    mpc = jnp.zeros((NP, NORB), _f32)
    b_rad2 = b_rad.reshape(1, NWAVE)
    for i in range(ITER_LOOP):
        acc = _sc_edge0(p, ect, idx_n, idx_c) if i == 0 else \
            _sc_edge(p, ect, idx_n, idx_c)
        dens, mpc, icf, p = _tc_node(acc, icf, dens, mpc, W_rad, b_rad2,
                                     W_msg[i], b_msg[i].reshape(1, NCOEF))
    acc = _sc_edge(p, ect, idx_n, idx_c)
    out = _tc_node_last(acc, icf, dens, mpc, W_rad, b_rad2, W_out)
    return out[0, 0] + N_NODES * b_out[0]


# speedup vs baseline: 77.6164x; 1.0001x over previous
import functools

import jax
import jax.numpy as jnp
import numpy as np
from jax import lax
from jax.experimental import pallas as pl
from jax.experimental.pallas import tpu as pltpu
from jax.experimental.pallas import tpu_sc as plsc

MAX_L = 2
NWAVE = 8
CUTOFF = 5.0
ITER_LOOP = 3
N_NODES = 50000
N_EDGES = 800000
NSPEC = 8
NORB = NWAVE * (MAX_L + 1)
NCOEF = 2 * (NWAVE + 1)

NP = 50176
EP = 802816
EPP = EP + 128
NC, NS, NW = 2, 16, 32
C = 128
ET = EP // NW
NCHUNK = ET // C
TROWS = NP // NS
ZB = 98
NZCP = TROWS // ZB

BN = 1024
GN = NP // BN
BE = 4096
GE = EP // BE

_f32 = jnp.float32


def _silu(x):
    return x * jax.nn.sigmoid(x)


def _full16(v):
    return jnp.full((16,), v, jnp.int32)


def _sc_setup_body(cart_hbm, idxn_hbm, idxc_hbm, sh_hbm, dv_hbm,
                   idxn_v, idxc_v, sh_v, gn_v, gc_v, dv_v, sem1, sem2):
    cid = lax.axis_index("c")
    sid = lax.axis_index("s")
    base = (sid * NC + cid) * ET

    def chunk(i, carry):
        e0 = base + i * C
        pltpu.sync_copy(idxn_hbm.at[pl.ds(e0, C)], idxn_v)
        pltpu.sync_copy(idxc_hbm.at[pl.ds(e0, C)], idxc_v)
        pltpu.sync_copy(sh_hbm.at[:, pl.ds(e0, C)], sh_v)
        cp1 = pltpu.async_copy(cart_hbm.at[idxn_v], gn_v, sem1)
        cp2 = pltpu.async_copy(cart_hbm.at[idxc_v], gc_v, sem2)
        cp1.wait()
        cp2.wait()
        for g in range(C // 16):
            rid = lax.iota(jnp.int32, 16) + g * 16
            r2 = jnp.zeros((16,), _f32)
            for j in range(3):
                xn = plsc.load_gather(gn_v, [rid, _full16(j)])
                xc = plsc.load_gather(gc_v, [rid, _full16(j)])
                dj = xn - xc + sh_v[j, pl.ds(g * 16, 16)]
                dv_v[j, pl.ds(g * 16, 16)] = dj
                r2 = r2 + dj * dj
            dv_v[3, pl.ds(g * 16, 16)] = r2
        pltpu.sync_copy(dv_v, dv_hbm.at[:, pl.ds(e0, C)])
        return carry

    lax.fori_loop(0, NCHUNK, chunk, 0)


def _sc_setup(cartp, idxn, idxc, shifts_t):
    mesh = plsc.VectorSubcoreMesh(core_axis_name="c", subcore_axis_name="s")
    f = pl.kernel(
        _sc_setup_body,
        out_type=jax.ShapeDtypeStruct((4, EP), _f32),
        mesh=mesh,
        compiler_params=pltpu.CompilerParams(needs_layout_passes=False, use_tc_tiling_on_sc=False),
        scratch_types=[
            pltpu.VMEM((C,), jnp.int32),
            pltpu.VMEM((C,), jnp.int32),
            pltpu.VMEM((4, C), _f32),
            pltpu.VMEM((C, 8), _f32),
            pltpu.VMEM((C, 8), _f32),
            pltpu.VMEM((4, C), _f32),
            pltpu.SemaphoreType.DMA,
            pltpu.SemaphoreType.DMA,
        ],
    )
    return f(cartp, idxn, idxc, shifts_t)


def _sc_edge_body(with_mpc, p_hbm, ec_hbm, idxn_hbm, idxc_hbm, out_hbm,
                  idxn_v0, idxn_v1, idxc_v0, idxc_v1, ec_v0, ec_v1,
                  rows_v0, rows_v1, out_v0, out_v1, zb_v, acc_sh,
                  asem0, asem1, gsem0, gsem1, ssem0, ssem1):
    cid = lax.axis_index("c")
    sid = lax.axis_index("s")
    idxn_v = (idxn_v0, idxn_v1)
    idxc_v = (idxc_v0, idxc_v1)
    ec_v = (ec_v0, ec_v1)
    rows_v = (rows_v0, rows_v1)
    out_v = (out_v0, out_v1)
    asem = (asem0, asem1)
    gsem = (gsem0, gsem1)
    ssem = (ssem0, ssem1)

    def zrow(r, carry):
        zb_v[r, pl.ds(0, 16)] = jnp.zeros((16,), _f32)
        zb_v[r, pl.ds(16, 16)] = jnp.zeros((16,), _f32)
        return carry

    lax.fori_loop(0, ZB, zrow, 0)

    def zcp(i, carry):
        pltpu.sync_copy(zb_v, acc_sh.at[pl.ds(sid * TROWS + i * ZB, ZB)])
        return carry

    lax.fori_loop(0, NZCP, zcp, 0)
    plsc.subcore_barrier()

    base = (sid * NC + cid) * ET

    def issue_a(c, nb):
        e1 = base + c * C
        a1 = pltpu.async_copy(idxn_hbm.at[pl.ds(e1, C)], idxn_v[nb], asem[nb])
        a2 = pltpu.async_copy(idxc_hbm.at[pl.ds(e1, C)], idxc_v[nb], asem[nb])
        a3 = pltpu.async_copy(ec_hbm.at[:, pl.ds(e1, C)], ec_v[nb], asem[nb])
        return a1, a2, a3

    def compute(b):
        def group2(g2, carry):
            for gg in range(2):
                rid = lax.iota(jnp.int32, 16) + (g2 * 2 + gg) * 16
                cut = plsc.load_gather(ec_v[b], [_full16(0), rid])
                cx = plsc.load_gather(ec_v[b], [_full16(1), rid])
                cy = plsc.load_gather(ec_v[b], [_full16(2), rid])
                cz = plsc.load_gather(ec_v[b], [_full16(3), rid])
                for k in range(NWAVE):
                    nck = plsc.load_gather(rows_v[b], [rid, _full16(k)])
                    plsc.store_scatter(out_v[b], [rid, _full16(k)], cut * nck)
                for k in range(NWAVE):
                    nc2 = plsc.load_gather(rows_v[b], [rid, _full16(8 + k)])
                    for j, cj in enumerate((cx, cy, cz)):
                        if with_mpc:
                            mpcv = plsc.load_gather(
                                rows_v[b], [rid, _full16(16 + j * 8 + k)])
                            val = cj * nc2 + mpcv
                        else:
                            val = cj * nc2
                        plsc.store_scatter(out_v[b],
                                           [rid, _full16(8 + j * 8 + k)], val)
            return carry

        lax.fori_loop(0, C // 32, group2, 0)

    def wait_scatter(b):
        pltpu.make_async_copy(out_v[b], acc_sh.at[idxc_v[b]], ssem[b]).wait()

    def wait_gather(b):
        pltpu.make_async_copy(p_hbm.at[idxn_v[b]], rows_v[b], gsem[b]).wait()

    def step(c, b, nb, first):
        if first:
            @pl.when(c >= 1)
            def _w():
                wait_scatter(nb)
        else:
            wait_scatter(nb)
        a1, a2, a3 = issue_a(c + 1, nb)
        wait_gather(b)
        compute(b)
        pltpu.async_copy(out_v[b], acc_sh.at[idxc_v[b]], ssem[b], add=True)
        a1.wait()
        a2.wait()
        a3.wait()
        pltpu.async_copy(p_hbm.at[idxn_v[nb]], rows_v[nb], gsem[nb])

    pltpu.sync_copy(idxn_hbm.at[pl.ds(base, C)], idxn_v[0])
    pltpu.sync_copy(idxc_hbm.at[pl.ds(base, C)], idxc_v[0])
    pltpu.sync_copy(ec_hbm.at[:, pl.ds(base, C)], ec_v[0])
    pltpu.async_copy(p_hbm.at[idxn_v[0]], rows_v[0], gsem[0])

    def pair(i2, carry):
        c0 = i2 * 2
        step(c0, 0, 1, True)
        step(c0 + 1, 1, 0, False)
        return carry

    lax.fori_loop(0, NCHUNK // 2, pair, 0)
    wait_scatter(1)
    wait_gather(0)
    plsc.subcore_barrier()
    r0 = sid * TROWS
    pltpu.sync_copy(acc_sh.at[pl.ds(r0, TROWS)],
                    out_hbm.at[cid, pl.ds(r0, TROWS)])


def _sc_edge_call(p, ect, idxn, idxc, pw, with_mpc):
    mesh = plsc.VectorSubcoreMesh(core_axis_name="c", subcore_axis_name="s")
    f = pl.kernel(
        functools.partial(_sc_edge_body, with_mpc),
        out_type=jax.ShapeDtypeStruct((NC, NP, 32), _f32),
        mesh=mesh,
        compiler_params=pltpu.CompilerParams(needs_layout_passes=False, use_tc_tiling_on_sc=False),
        scratch_types=[
            pltpu.VMEM((C,), jnp.int32),
            pltpu.VMEM((C,), jnp.int32),
            pltpu.VMEM((C,), jnp.int32),
            pltpu.VMEM((C,), jnp.int32),
            pltpu.VMEM((4, C), _f32),
            pltpu.VMEM((4, C), _f32),
            pltpu.VMEM((C, pw), _f32),
            pltpu.VMEM((C, pw), _f32),
            pltpu.VMEM((C, 32), _f32),
            pltpu.VMEM((C, 32), _f32),
            pltpu.VMEM((ZB, 32), _f32),
            pltpu.VMEM_SHARED((NP, 32), _f32),
            pltpu.SemaphoreType.DMA,
            pltpu.SemaphoreType.DMA,
            pltpu.SemaphoreType.DMA,
            pltpu.SemaphoreType.DMA,
            pltpu.SemaphoreType.DMA,
            pltpu.SemaphoreType.DMA,
        ],
    )
    return f(p, ect, idxn, idxc)


def _sc_edge(p, ect, idxn, idxc):
    return _sc_edge_call(p, ect, idxn, idxc, 40, True)


def _sc_edge0(p0, ect, idxn, idxc):
    return _sc_edge_call(p0, ect, idxn, idxc, 16, False)


def _tc_prep_body(dv_ref, ec_ref):
    pid = pl.program_id(0)
    dv = dv_ref[...]
    dx = dv[0:1, :]
    dy = dv[1:2, :]
    dz = dv[2:3, :]
    r2 = dv[3:4, :]
    d = jnp.sqrt(r2)
    cut = jnp.power(0.5 * jnp.cos(d * (np.pi / CUTOFF)) + 0.5, 3)
    col = lax.broadcasted_iota(jnp.int32, (1, BE), 1) + pid * BE
    cut = jnp.where(col < N_EDGES, cut, 0.0)
    ec_ref[...] = jnp.concatenate([cut, cut * dx, cut * dy, cut * dz], axis=0)


def _tc_prep(dvt):
    return pl.pallas_call(
        _tc_prep_body,
        grid=(GE,),
        in_specs=[pl.BlockSpec((4, BE), lambda i: (0, i))],
        out_specs=pl.BlockSpec((4, BE), lambda i: (0, i)),
        out_shape=jax.ShapeDtypeStruct((4, EP), _f32),
    )(dvt)


def _tc_emb_body(sp_ref, we_ref, be_ref, icf_ref, p_ref):
    pid = pl.program_id(0)
    icf = _silu(jnp.dot(sp_ref[...], we_ref[...],
                        preferred_element_type=_f32) + be_ref[...])
    icf_ref[...] = icf
    rows = lax.broadcasted_iota(jnp.int32, (BN, 1), 0) + pid * BN
    mask = (rows < N_NODES).astype(_f32)
    p = jnp.concatenate([icf[:, 0:8], icf[:, 9:17]], axis=1)
    p_ref[...] = p * mask


def _tc_emb(speciesp, w_emb, b_emb):
    return pl.pallas_call(
        _tc_emb_body,
        grid=(GN,),
        in_specs=[
            pl.BlockSpec((BN, NSPEC), lambda i: (i, 0)),
            pl.BlockSpec((NSPEC, NCOEF), lambda i: (0, 0)),
            pl.BlockSpec((1, NCOEF), lambda i: (0, 0)),
        ],
        out_specs=[
            pl.BlockSpec((BN, NCOEF), lambda i: (i, 0)),
            pl.BlockSpec((BN, 16), lambda i: (i, 0)),
        ],
        out_shape=[
            jax.ShapeDtypeStruct((NP, NCOEF), _f32),
            jax.ShapeDtypeStruct((NP, 16), _f32),
        ],
    )(speciesp, w_emb, b_emb)


def _tc_node_body(acc0_ref, acc1_ref, icf_ref, dens_ref, mpc_ref,
                  wr_ref, br_ref, wm_ref, bm_ref,
                  dens_out, mpc_out, icf_out, p_out):
    pid = pl.program_id(0)
    acc = acc0_ref[0] + acc1_ref[0]
    mpd = acc[:, 0:8]
    ss = acc[:, 8:32]
    icf = icf_ref[...]
    mpc = mpc_ref[...] + ss
    radial = _silu(jnp.dot(mpd * icf[:, 8:9], wr_ref[...],
                           preferred_element_type=_f32) + br_ref[...])
    c = icf[:, 17:18]
    x = mpc[:, 0:8] * c
    y = mpc[:, 8:16] * c
    z = mpc[:, 16:24] * c
    r2 = x * x + y * y + z * z
    ang2 = ((x * y) ** 2 + (y * z) ** 2 + (3.0 * z * z - r2) ** 2
            + (x * z) ** 2 + (x * x - y * y) ** 2)
    dens = dens_ref[...] + jnp.concatenate(
        [radial, radial * r2, radial * ang2], axis=1)
    dens_out[...] = dens
    mpc_out[...] = mpc
    icf_new = _silu(jnp.dot(dens, wm_ref[...],
                            preferred_element_type=_f32) + bm_ref[...])
    icf_out[...] = icf_new
    rows = lax.broadcasted_iota(jnp.int32, (BN, 1), 0) + pid * BN
    mask = (rows < N_NODES).astype(_f32)
    p = jnp.concatenate([icf_new[:, 0:8], icf_new[:, 9:17], mpc], axis=1)
    p_out[...] = p * mask


def _tc_node(acc, icf, dens, mpc, w_rad, b_rad, w_msg_i, b_msg_i):
    return pl.pallas_call(
        _tc_node_body,
        grid=(GN,),
        in_specs=[
            pl.BlockSpec((1, BN, 32), lambda i: (0, i, 0)),
            pl.BlockSpec((1, BN, 32), lambda i: (1, i, 0)),
            pl.BlockSpec((BN, NCOEF), lambda i: (i, 0)),
            pl.BlockSpec((BN, NORB), lambda i: (i, 0)),
            pl.BlockSpec((BN, NORB), lambda i: (i, 0)),
            pl.BlockSpec((NWAVE, NWAVE), lambda i: (0, 0)),
            pl.BlockSpec((1, NWAVE), lambda i: (0, 0)),
            pl.BlockSpec((NORB, NCOEF), lambda i: (0, 0)),
            pl.BlockSpec((1, NCOEF), lambda i: (0, 0)),
        ],
        out_specs=[
            pl.BlockSpec((BN, NORB), lambda i: (i, 0)),
            pl.BlockSpec((BN, NORB), lambda i: (i, 0)),
            pl.BlockSpec((BN, NCOEF), lambda i: (i, 0)),
            pl.BlockSpec((BN, 40), lambda i: (i, 0)),
        ],
        out_shape=[
            jax.ShapeDtypeStruct((NP, NORB), _f32),
            jax.ShapeDtypeStruct((NP, NORB), _f32),
            jax.ShapeDtypeStruct((NP, NCOEF), _f32),
            jax.ShapeDtypeStruct((NP, 40), _f32),
        ],
    )(acc, acc, icf, dens, mpc, w_rad, b_rad, w_msg_i, b_msg_i)


def _tc_node_last_body(acc0_ref, acc1_ref, icf_ref, dens_ref, mpc_ref,
                       wr_ref, br_ref, wo_ref, out_ref):
    pid = pl.program_id(0)
    acc = acc0_ref[0] + acc1_ref[0]
    mpd = acc[:, 0:8]
    ss = acc[:, 8:32]
    icf = icf_ref[...]
    mpc = mpc_ref[...] + ss
    radial = _silu(jnp.dot(mpd * icf[:, 8:9], wr_ref[...],
                           preferred_element_type=_f32) + br_ref[...])
    c = icf[:, 17:18]
    x = mpc[:, 0:8] * c
    y = mpc[:, 8:16] * c
    z = mpc[:, 16:24] * c
    r2 = x * x + y * y + z * z
    ang2 = ((x * y) ** 2 + (y * z) ** 2 + (3.0 * z * z - r2) ** 2
            + (x * z) ** 2 + (x * x - y * y) ** 2)
    dens = dens_ref[...] + jnp.concatenate(
        [radial, radial * r2, radial * ang2], axis=1)
    rows = lax.broadcasted_iota(jnp.int32, (BN, 1), 0) + pid * BN
    mask = (rows < N_NODES).astype(_f32)
    part = jnp.sum(jnp.dot(dens * mask, wo_ref[...],
                           preferred_element_type=_f32))

    @pl.when(pid == 0)
    def _init():
        out_ref[...] = jnp.zeros((1, 1), _f32)

    out_ref[...] = out_ref[...] + part


def _tc_node_last(acc, icf, dens, mpc, w_rad, b_rad, w_out):
    return pl.pallas_call(
        _tc_node_last_body,
        grid=(GN,),
        in_specs=[
            pl.BlockSpec((1, BN, 32), lambda i: (0, i, 0)),
            pl.BlockSpec((1, BN, 32), lambda i: (1, i, 0)),
            pl.BlockSpec((BN, NCOEF), lambda i: (i, 0)),
            pl.BlockSpec((BN, NORB), lambda i: (i, 0)),
            pl.BlockSpec((BN, NORB), lambda i: (i, 0)),
            pl.BlockSpec((NWAVE, NWAVE), lambda i: (0, 0)),
            pl.BlockSpec((1, NWAVE), lambda i: (0, 0)),
            pl.BlockSpec((NORB, 1), lambda i: (0, 0)),
        ],
        out_specs=pl.BlockSpec((1, 1), lambda i: (0, 0)),
        out_shape=jax.ShapeDtypeStruct((1, 1), _f32),
    )(acc, acc, icf, dens, mpc, w_rad, b_rad, w_out)


def kernel(cart, neighlist, shifts, species, W_emb, b_emb, W_rad, b_rad,
           W_msg, b_msg, W_out, b_out):
    idx_c = neighlist[0].astype(jnp.int32)
    idx_n = neighlist[1].astype(jnp.int32)
    idx_c = jnp.pad(idx_c, (0, EPP - N_EDGES), constant_values=N_NODES)
    idx_n = jnp.pad(idx_n, (0, EPP - N_EDGES), constant_values=N_NODES)
    shifts_t = jnp.pad(shifts.T.astype(_f32), ((0, 1), (0, EP - N_EDGES)))
    cartp = jnp.pad(cart.astype(_f32), ((0, NP - N_NODES), (0, 5)))
    speciesp = jnp.pad(species.astype(_f32), ((0, NP - N_NODES), (0, 0)))

    dvt = _sc_setup(cartp, idx_n, idx_c, shifts_t)
    ect = jnp.pad(_tc_prep(dvt), ((0, 0), (0, EPP - EP)))
    icf, p = _tc_emb(speciesp, W_emb, b_emb.reshape(1, NCOEF))

    dens = jnp.zeros((NP, NORB), _f32)
    mpc = jnp.zeros((NP, NORB), _f32)
    b_rad2 = b_rad.reshape(1, NWAVE)
    for i in range(ITER_LOOP):
        acc = _sc_edge0(p, ect, idx_n, idx_c) if i == 0 else \
            _sc_edge(p, ect, idx_n, idx_c)
        dens, mpc, icf, p = _tc_node(acc, icf, dens, mpc, W_rad, b_rad2,
                                     W_msg[i], b_msg[i].reshape(1, NCOEF))
    acc = _sc_edge(p, ect, idx_n, idx_c)
    out = _tc_node_last(acc, icf, dens, mpc, W_rad, b_rad2, W_out)
    return out[0, 0] + N_NODES * b_out[0]


# speedup vs baseline: 92.5676x; 1.1926x over previous
import functools

import jax
import jax.numpy as jnp
import numpy as np
from jax import lax
from jax.experimental import pallas as pl
from jax.experimental.pallas import tpu as pltpu
from jax.experimental.pallas import tpu_sc as plsc

MAX_L = 2
NWAVE = 8
CUTOFF = 5.0
ITER_LOOP = 3
N_NODES = 50000
N_EDGES = 800000
NSPEC = 8
NORB = NWAVE * (MAX_L + 1)
NCOEF = 2 * (NWAVE + 1)

NP = 50176
EP = 802816
EPP = EP + 256
NC, NS, NW = 2, 16, 32
C = 128
ET = EP // NW
NCHUNK = ET // C
TROWS = NP // NS
ZB = 98
NZCP = TROWS // ZB

BN = 1024
GN = NP // BN
BE = 4096
GE = EP // BE

_f32 = jnp.float32


def _silu(x):
    return x * jax.nn.sigmoid(x)


def _full16(v):
    return jnp.full((16,), v, jnp.int32)


def _sc_setup_body(cart_hbm, idxn_hbm, idxc_hbm, sh_hbm, dv_hbm,
                   idxn_v, idxc_v, sh_v, gn_v, gc_v, dv_v, sem1, sem2):
    cid = lax.axis_index("c")
    sid = lax.axis_index("s")
    base = (sid * NC + cid) * ET

    def chunk(i, carry):
        e0 = base + i * C
        pltpu.sync_copy(idxn_hbm.at[pl.ds(e0, C)], idxn_v)
        pltpu.sync_copy(idxc_hbm.at[pl.ds(e0, C)], idxc_v)
        pltpu.sync_copy(sh_hbm.at[:, pl.ds(e0, C)], sh_v)
        cp1 = pltpu.async_copy(cart_hbm.at[idxn_v], gn_v, sem1)
        cp2 = pltpu.async_copy(cart_hbm.at[idxc_v], gc_v, sem2)
        cp1.wait()
        cp2.wait()
        for g in range(C // 16):
            rid = lax.iota(jnp.int32, 16) + g * 16
            r2 = jnp.zeros((16,), _f32)
            for j in range(3):
                xn = plsc.load_gather(gn_v, [rid, _full16(j)])
                xc = plsc.load_gather(gc_v, [rid, _full16(j)])
                dj = xn - xc + sh_v[j, pl.ds(g * 16, 16)]
                dv_v[j, pl.ds(g * 16, 16)] = dj
                r2 = r2 + dj * dj
            dv_v[3, pl.ds(g * 16, 16)] = r2
        pltpu.sync_copy(dv_v, dv_hbm.at[:, pl.ds(e0, C)])
        return carry

    lax.fori_loop(0, NCHUNK, chunk, 0)


def _sc_setup(cartp, idxn, idxc, shifts_t):
    mesh = plsc.VectorSubcoreMesh(core_axis_name="c", subcore_axis_name="s")
    f = pl.kernel(
        _sc_setup_body,
        out_type=jax.ShapeDtypeStruct((4, EP), _f32),
        mesh=mesh,
        compiler_params=pltpu.CompilerParams(needs_layout_passes=False, use_tc_tiling_on_sc=False),
        scratch_types=[
            pltpu.VMEM((C,), jnp.int32),
            pltpu.VMEM((C,), jnp.int32),
            pltpu.VMEM((4, C), _f32),
            pltpu.VMEM((C, 8), _f32),
            pltpu.VMEM((C, 8), _f32),
            pltpu.VMEM((4, C), _f32),
            pltpu.SemaphoreType.DMA,
            pltpu.SemaphoreType.DMA,
        ],
    )
    return f(cartp, idxn, idxc, shifts_t)


def _sc_edge_body(with_mpc, p_hbm, ec_hbm, idxn_hbm, idxc_hbm, out_hbm,
                  idxn_v0, idxn_v1, idxn_v2, idxn_v3,
                  idxc_v0, idxc_v1, idxc_v2, idxc_v3,
                  ec_v0, ec_v1, ec_v2, ec_v3,
                  rows_v0, rows_v1, out_v0, out_v1, zb_v, acc_sh,
                  asem0, asem1, asem2, asem3, gsem0, gsem1, ssem0, ssem1):
    cid = lax.axis_index("c")
    sid = lax.axis_index("s")
    idxn_v = (idxn_v0, idxn_v1, idxn_v2, idxn_v3)
    idxc_v = (idxc_v0, idxc_v1, idxc_v2, idxc_v3)
    ec_v = (ec_v0, ec_v1, ec_v2, ec_v3)
    rows_v = (rows_v0, rows_v1)
    out_v = (out_v0, out_v1)
    asem = (asem0, asem1, asem2, asem3)
    gsem = (gsem0, gsem1)
    ssem = (ssem0, ssem1)

    def zrow(r, carry):
        zb_v[r, pl.ds(0, 16)] = jnp.zeros((16,), _f32)
        zb_v[r, pl.ds(16, 16)] = jnp.zeros((16,), _f32)
        return carry

    lax.fori_loop(0, ZB, zrow, 0)

    def zcp(i, carry):
        pltpu.sync_copy(zb_v, acc_sh.at[pl.ds(sid * TROWS + i * ZB, ZB)])
        return carry

    lax.fori_loop(0, NZCP, zcp, 0)
    plsc.subcore_barrier()

    base = (sid * NC + cid) * ET

    def issue_streams(c, q):
        e1 = base + c * C
        pltpu.async_copy(idxn_hbm.at[pl.ds(e1, C)], idxn_v[q], asem[q])
        pltpu.async_copy(idxc_hbm.at[pl.ds(e1, C)], idxc_v[q], asem[q])
        pltpu.async_copy(ec_hbm.at[:, pl.ds(e1, C)], ec_v[q], asem[q])

    def wait_streams(q):
        pltpu.make_async_copy(idxn_hbm.at[pl.ds(base, C)], idxn_v[q],
                              asem[q]).wait()
        pltpu.make_async_copy(idxc_hbm.at[pl.ds(base, C)], idxc_v[q],
                              asem[q]).wait()
        pltpu.make_async_copy(ec_hbm.at[:, pl.ds(base, C)], ec_v[q],
                              asem[q]).wait()

    def compute(b, q):
        def group2(g2, carry):
            for gg in range(2):
                rid = lax.iota(jnp.int32, 16) + (g2 * 2 + gg) * 16
                cut = plsc.load_gather(ec_v[q], [_full16(0), rid])
                cx = plsc.load_gather(ec_v[q], [_full16(1), rid])
                cy = plsc.load_gather(ec_v[q], [_full16(2), rid])
                cz = plsc.load_gather(ec_v[q], [_full16(3), rid])
                for k in range(NWAVE):
                    nck = plsc.load_gather(rows_v[b], [rid, _full16(k)])
                    plsc.store_scatter(out_v[b], [rid, _full16(k)], cut * nck)
                for k in range(NWAVE):
                    nc2 = plsc.load_gather(rows_v[b], [rid, _full16(8 + k)])
                    for j, cj in enumerate((cx, cy, cz)):
                        if with_mpc:
                            mpcv = plsc.load_gather(
                                rows_v[b], [rid, _full16(16 + j * 8 + k)])
                            val = cj * nc2 + mpcv
                        else:
                            val = cj * nc2
                        plsc.store_scatter(out_v[b],
                                           [rid, _full16(8 + j * 8 + k)], val)
            return carry

        lax.fori_loop(0, C // 32, group2, 0)

    def wait_scatter(b, q):
        pltpu.make_async_copy(out_v[b], acc_sh.at[idxc_v[q]], ssem[b]).wait()

    def wait_gather(b, q):
        pltpu.make_async_copy(p_hbm.at[idxn_v[q]], rows_v[b], gsem[b]).wait()

    def step(c, q, q1, q2, q3, b, nb, first):
        wait_streams(q1)
        pltpu.async_copy(p_hbm.at[idxn_v[q1]], rows_v[nb], gsem[nb])
        issue_streams(c + 2, q2)
        wait_gather(b, q)
        compute(b, q)
        if first:
            @pl.when(c >= 1)
            def _w():
                wait_scatter(nb, q3)
        else:
            wait_scatter(nb, q3)
        pltpu.async_copy(out_v[b], acc_sh.at[idxc_v[q]], ssem[b], add=True)

    pltpu.sync_copy(idxn_hbm.at[pl.ds(base, C)], idxn_v[0])
    pltpu.sync_copy(idxc_hbm.at[pl.ds(base, C)], idxc_v[0])
    pltpu.sync_copy(ec_hbm.at[:, pl.ds(base, C)], ec_v[0])
    issue_streams(1, 1)
    pltpu.async_copy(p_hbm.at[idxn_v[0]], rows_v[0], gsem[0])

    def quad(i4, carry):
        c0 = i4 * 4
        step(c0, 0, 1, 2, 3, 0, 1, True)
        step(c0 + 1, 1, 2, 3, 0, 1, 0, False)
        step(c0 + 2, 2, 3, 0, 1, 0, 1, False)
        step(c0 + 3, 3, 0, 1, 2, 1, 0, False)
        return carry

    lax.fori_loop(0, NCHUNK // 4, quad, 0)
    wait_streams(1)
    wait_gather(0, 0)
    wait_scatter(1, 3)
    plsc.subcore_barrier()
    r0 = sid * TROWS
    pltpu.sync_copy(acc_sh.at[pl.ds(r0, TROWS)],
                    out_hbm.at[cid, pl.ds(r0, TROWS)])


def _sc_edge_call(p, ect, idxn, idxc, pw, with_mpc):
    mesh = plsc.VectorSubcoreMesh(core_axis_name="c", subcore_axis_name="s")
    f = pl.kernel(
        functools.partial(_sc_edge_body, with_mpc),
        out_type=jax.ShapeDtypeStruct((NC, NP, 32), _f32),
        mesh=mesh,
        compiler_params=pltpu.CompilerParams(needs_layout_passes=False, use_tc_tiling_on_sc=False),
        scratch_types=(
            [pltpu.VMEM((C,), jnp.int32)] * 8
            + [pltpu.VMEM((4, C), _f32)] * 4
            + [pltpu.VMEM((C, pw), _f32)] * 2
            + [pltpu.VMEM((C, 32), _f32)] * 2
            + [pltpu.VMEM((ZB, 32), _f32)]
            + [pltpu.VMEM_SHARED((NP, 32), _f32)]
            + [pltpu.SemaphoreType.DMA] * 8
        ),
    )
    return f(p, ect, idxn, idxc)


def _sc_edge(p, ect, idxn, idxc):
    return _sc_edge_call(p, ect, idxn, idxc, 40, True)


def _sc_edge0(p0, ect, idxn, idxc):
    return _sc_edge_call(p0, ect, idxn, idxc, 16, False)


def _tc_prep_body(dv_ref, ec_ref):
    pid = pl.program_id(0)
    dv = dv_ref[...]
    dx = dv[0:1, :]
    dy = dv[1:2, :]
    dz = dv[2:3, :]
    r2 = dv[3:4, :]
    d = jnp.sqrt(r2)
    cut = jnp.power(0.5 * jnp.cos(d * (np.pi / CUTOFF)) + 0.5, 3)
    col = lax.broadcasted_iota(jnp.int32, (1, BE), 1) + pid * BE
    cut = jnp.where(col < N_EDGES, cut, 0.0)
    ec_ref[...] = jnp.concatenate([cut, cut * dx, cut * dy, cut * dz], axis=0)


def _tc_prep(dvt):
    return pl.pallas_call(
        _tc_prep_body,
        grid=(GE,),
        in_specs=[pl.BlockSpec((4, BE), lambda i: (0, i))],
        out_specs=pl.BlockSpec((4, BE), lambda i: (0, i)),
        out_shape=jax.ShapeDtypeStruct((4, EP), _f32),
    )(dvt)


def _tc_emb_body(sp_ref, we_ref, be_ref, icf_ref, p_ref):
    pid = pl.program_id(0)
    icf = _silu(jnp.dot(sp_ref[...], we_ref[...],
                        preferred_element_type=_f32) + be_ref[...])
    icf_ref[...] = icf
    rows = lax.broadcasted_iota(jnp.int32, (BN, 1), 0) + pid * BN
    mask = (rows < N_NODES).astype(_f32)
    p = jnp.concatenate([icf[:, 0:8], icf[:, 9:17]], axis=1)
    p_ref[...] = p * mask


def _tc_emb(speciesp, w_emb, b_emb):
    return pl.pallas_call(
        _tc_emb_body,
        grid=(GN,),
        in_specs=[
            pl.BlockSpec((BN, NSPEC), lambda i: (i, 0)),
            pl.BlockSpec((NSPEC, NCOEF), lambda i: (0, 0)),
            pl.BlockSpec((1, NCOEF), lambda i: (0, 0)),
        ],
        out_specs=[
            pl.BlockSpec((BN, NCOEF), lambda i: (i, 0)),
            pl.BlockSpec((BN, 16), lambda i: (i, 0)),
        ],
        out_shape=[
            jax.ShapeDtypeStruct((NP, NCOEF), _f32),
            jax.ShapeDtypeStruct((NP, 16), _f32),
        ],
    )(speciesp, w_emb, b_emb)


def _tc_node_body(acc0_ref, acc1_ref, icf_ref, dens_ref, mpc_ref,
                  wr_ref, br_ref, wm_ref, bm_ref,
                  dens_out, mpc_out, icf_out, p_out):
    pid = pl.program_id(0)
    acc = acc0_ref[0] + acc1_ref[0]
    mpd = acc[:, 0:8]
    ss = acc[:, 8:32]
    icf = icf_ref[...]
    mpc = mpc_ref[...] + ss
    radial = _silu(jnp.dot(mpd * icf[:, 8:9], wr_ref[...],
                           preferred_element_type=_f32) + br_ref[...])
    c = icf[:, 17:18]
    x = mpc[:, 0:8] * c
    y = mpc[:, 8:16] * c
    z = mpc[:, 16:24] * c
    r2 = x * x + y * y + z * z
    ang2 = ((x * y) ** 2 + (y * z) ** 2 + (3.0 * z * z - r2) ** 2
            + (x * z) ** 2 + (x * x - y * y) ** 2)
    dens = dens_ref[...] + jnp.concatenate(
        [radial, radial * r2, radial * ang2], axis=1)
    dens_out[...] = dens
    mpc_out[...] = mpc
    icf_new = _silu(jnp.dot(dens, wm_ref[...],
                            preferred_element_type=_f32) + bm_ref[...])
    icf_out[...] = icf_new
    rows = lax.broadcasted_iota(jnp.int32, (BN, 1), 0) + pid * BN
    mask = (rows < N_NODES).astype(_f32)
    p = jnp.concatenate([icf_new[:, 0:8], icf_new[:, 9:17], mpc], axis=1)
    p_out[...] = p * mask


def _tc_node(acc, icf, dens, mpc, w_rad, b_rad, w_msg_i, b_msg_i):
    return pl.pallas_call(
        _tc_node_body,
        grid=(GN,),
        in_specs=[
            pl.BlockSpec((1, BN, 32), lambda i: (0, i, 0)),
            pl.BlockSpec((1, BN, 32), lambda i: (1, i, 0)),
            pl.BlockSpec((BN, NCOEF), lambda i: (i, 0)),
            pl.BlockSpec((BN, NORB), lambda i: (i, 0)),
            pl.BlockSpec((BN, NORB), lambda i: (i, 0)),
            pl.BlockSpec((NWAVE, NWAVE), lambda i: (0, 0)),
            pl.BlockSpec((1, NWAVE), lambda i: (0, 0)),
            pl.BlockSpec((NORB, NCOEF), lambda i: (0, 0)),
            pl.BlockSpec((1, NCOEF), lambda i: (0, 0)),
        ],
        out_specs=[
            pl.BlockSpec((BN, NORB), lambda i: (i, 0)),
            pl.BlockSpec((BN, NORB), lambda i: (i, 0)),
            pl.BlockSpec((BN, NCOEF), lambda i: (i, 0)),
            pl.BlockSpec((BN, 40), lambda i: (i, 0)),
        ],
        out_shape=[
            jax.ShapeDtypeStruct((NP, NORB), _f32),
            jax.ShapeDtypeStruct((NP, NORB), _f32),
            jax.ShapeDtypeStruct((NP, NCOEF), _f32),
            jax.ShapeDtypeStruct((NP, 40), _f32),
        ],
    )(acc, acc, icf, dens, mpc, w_rad, b_rad, w_msg_i, b_msg_i)


def _tc_node_last_body(acc0_ref, acc1_ref, icf_ref, dens_ref, mpc_ref,
                       wr_ref, br_ref, wo_ref, out_ref):
    pid = pl.program_id(0)
    acc = acc0_ref[0] + acc1_ref[0]
    mpd = acc[:, 0:8]
    ss = acc[:, 8:32]
    icf = icf_ref[...]
    mpc = mpc_ref[...] + ss
    radial = _silu(jnp.dot(mpd * icf[:, 8:9], wr_ref[...],
                           preferred_element_type=_f32) + br_ref[...])
    c = icf[:, 17:18]
    x = mpc[:, 0:8] * c
    y = mpc[:, 8:16] * c
    z = mpc[:, 16:24] * c
    r2 = x * x + y * y + z * z
    ang2 = ((x * y) ** 2 + (y * z) ** 2 + (3.0 * z * z - r2) ** 2
            + (x * z) ** 2 + (x * x - y * y) ** 2)
    dens = dens_ref[...] + jnp.concatenate(
        [radial, radial * r2, radial * ang2], axis=1)
    rows = lax.broadcasted_iota(jnp.int32, (BN, 1), 0) + pid * BN
    mask = (rows < N_NODES).astype(_f32)
    part = jnp.sum(jnp.dot(dens * mask, wo_ref[...],
                           preferred_element_type=_f32))

    @pl.when(pid == 0)
    def _init():
        out_ref[...] = jnp.zeros((1, 1), _f32)

    out_ref[...] = out_ref[...] + part


def _tc_node_last(acc, icf, dens, mpc, w_rad, b_rad, w_out):
    return pl.pallas_call(
        _tc_node_last_body,
        grid=(GN,),
        in_specs=[
            pl.BlockSpec((1, BN, 32), lambda i: (0, i, 0)),
            pl.BlockSpec((1, BN, 32), lambda i: (1, i, 0)),
            pl.BlockSpec((BN, NCOEF), lambda i: (i, 0)),
            pl.BlockSpec((BN, NORB), lambda i: (i, 0)),
            pl.BlockSpec((BN, NORB), lambda i: (i, 0)),
            pl.BlockSpec((NWAVE, NWAVE), lambda i: (0, 0)),
            pl.BlockSpec((1, NWAVE), lambda i: (0, 0)),
            pl.BlockSpec((NORB, 1), lambda i: (0, 0)),
        ],
        out_specs=pl.BlockSpec((1, 1), lambda i: (0, 0)),
        out_shape=jax.ShapeDtypeStruct((1, 1), _f32),
    )(acc, acc, icf, dens, mpc, w_rad, b_rad, w_out)


def kernel(cart, neighlist, shifts, species, W_emb, b_emb, W_rad, b_rad,
           W_msg, b_msg, W_out, b_out):
    idx_c = neighlist[0].astype(jnp.int32)
    idx_n = neighlist[1].astype(jnp.int32)
    idx_c = jnp.pad(idx_c, (0, EPP - N_EDGES), constant_values=N_NODES)
    idx_n = jnp.pad(idx_n, (0, EPP - N_EDGES), constant_values=N_NODES)
    shifts_t = jnp.pad(shifts.T.astype(_f32), ((0, 1), (0, EP - N_EDGES)))
    cartp = jnp.pad(cart.astype(_f32), ((0, NP - N_NODES), (0, 5)))
    speciesp = jnp.pad(species.astype(_f32), ((0, NP - N_NODES), (0, 0)))

    dvt = _sc_setup(cartp, idx_n, idx_c, shifts_t)
    ect = jnp.pad(_tc_prep(dvt), ((0, 0), (0, EPP - EP)))
    icf, p = _tc_emb(speciesp, W_emb, b_emb.reshape(1, NCOEF))

    dens = jnp.zeros((NP, NORB), _f32)
    mpc = jnp.zeros((NP, NORB), _f32)
    b_rad2 = b_rad.reshape(1, NWAVE)
    for i in range(ITER_LOOP):
        acc = _sc_edge0(p, ect, idx_n, idx_c) if i == 0 else \
            _sc_edge(p, ect, idx_n, idx_c)
        dens, mpc, icf, p = _tc_node(acc, icf, dens, mpc, W_rad, b_rad2,
                                     W_msg[i], b_msg[i].reshape(1, NCOEF))
    acc = _sc_edge(p, ect, idx_n, idx_c)
    out = _tc_node_last(acc, icf, dens, mpc, W_rad, b_rad2, W_out)
    return out[0, 0] + N_NODES * b_out[0]


# speedup vs baseline: 101.0665x; 1.0918x over previous
import functools

import jax
import jax.numpy as jnp
import numpy as np
from jax import lax
from jax.experimental import pallas as pl
from jax.experimental.pallas import tpu as pltpu
from jax.experimental.pallas import tpu_sc as plsc

MAX_L = 2
NWAVE = 8
CUTOFF = 5.0
ITER_LOOP = 3
N_NODES = 50000
N_EDGES = 800000
NSPEC = 8
NORB = NWAVE * (MAX_L + 1)
NCOEF = 2 * (NWAVE + 1)

NP = 50176
EP = 802816
EPP = EP + 256
NC, NS, NW = 2, 16, 32
C = 128
ET = EP // NW
NCHUNK = ET // C
TROWS = NP // NS
ZB = 98
NZCP = TROWS // ZB

BN = 1024
GN = NP // BN
BE = 4096
GE = EP // BE

_f32 = jnp.float32


def _silu(x):
    return x * jax.nn.sigmoid(x)


def _full16(v):
    return jnp.full((16,), v, jnp.int32)


def _sc_setup_body(cart_hbm, idxn_hbm, idxc_hbm, sh_hbm, dv_hbm,
                   idxn_v0, idxn_v1, idxn_v2, idxn_v3,
                   idxc_v0, idxc_v1, idxc_v2, idxc_v3,
                   sh_v0, sh_v1, sh_v2, sh_v3,
                   gn_v0, gn_v1, gc_v0, gc_v1, dv_v0, dv_v1,
                   asem0, asem1, asem2, asem3, gsem0, gsem1, dsem0, dsem1):
    cid = lax.axis_index("c")
    sid = lax.axis_index("s")
    idxn_v = (idxn_v0, idxn_v1, idxn_v2, idxn_v3)
    idxc_v = (idxc_v0, idxc_v1, idxc_v2, idxc_v3)
    sh_v = (sh_v0, sh_v1, sh_v2, sh_v3)
    gn_v = (gn_v0, gn_v1)
    gc_v = (gc_v0, gc_v1)
    dv_v = (dv_v0, dv_v1)
    asem = (asem0, asem1, asem2, asem3)
    gsem = (gsem0, gsem1)
    dsem = (dsem0, dsem1)
    base = (sid * NC + cid) * ET

    def issue_streams(c, q):
        e1 = base + c * C
        pltpu.async_copy(idxn_hbm.at[pl.ds(e1, C)], idxn_v[q], asem[q])
        pltpu.async_copy(idxc_hbm.at[pl.ds(e1, C)], idxc_v[q], asem[q])
        pltpu.async_copy(sh_hbm.at[:, pl.ds(e1, C)], sh_v[q], asem[q])

    def wait_streams(q):
        pltpu.make_async_copy(idxn_hbm.at[pl.ds(base, C)], idxn_v[q],
                              asem[q]).wait()
        pltpu.make_async_copy(idxc_hbm.at[pl.ds(base, C)], idxc_v[q],
                              asem[q]).wait()
        pltpu.make_async_copy(sh_hbm.at[:, pl.ds(base, C)], sh_v[q],
                              asem[q]).wait()

    def wait_gathers(b, q):
        pltpu.make_async_copy(cart_hbm.at[idxn_v[q]], gn_v[b],
                              gsem[b]).wait()
        pltpu.make_async_copy(cart_hbm.at[idxc_v[q]], gc_v[b],
                              gsem[b]).wait()

    def wait_store(c, b):
        pltpu.make_async_copy(dv_v[b], dv_hbm.at[:, pl.ds(base, C)],
                              dsem[b]).wait()

    def compute(b, q):
        for g in range(C // 16):
            rid = lax.iota(jnp.int32, 16) + g * 16
            r2 = jnp.zeros((16,), _f32)
            for j in range(3):
                xn = plsc.load_gather(gn_v[b], [rid, _full16(j)])
                xc = plsc.load_gather(gc_v[b], [rid, _full16(j)])
                dj = xn - xc + sh_v[q][j, pl.ds(g * 16, 16)]
                dv_v[b][j, pl.ds(g * 16, 16)] = dj
                r2 = r2 + dj * dj
            dv_v[b][3, pl.ds(g * 16, 16)] = r2

    def step(c, q, q1, q2, b, nb, first):
        wait_streams(q1)
        pltpu.async_copy(cart_hbm.at[idxn_v[q1]], gn_v[nb], gsem[nb])
        pltpu.async_copy(cart_hbm.at[idxc_v[q1]], gc_v[nb], gsem[nb])
        issue_streams(c + 2, q2)
        wait_gathers(b, q)
        if first:
            @pl.when(c >= 2)
            def _w():
                wait_store(c - 2, b)
        else:
            wait_store(c - 2, b)
        compute(b, q)
        pltpu.async_copy(dv_v[b], dv_hbm.at[:, pl.ds(base + c * C, C)],
                         dsem[b])

    pltpu.sync_copy(idxn_hbm.at[pl.ds(base, C)], idxn_v[0])
    pltpu.sync_copy(idxc_hbm.at[pl.ds(base, C)], idxc_v[0])
    pltpu.sync_copy(sh_hbm.at[:, pl.ds(base, C)], sh_v[0])
    issue_streams(1, 1)
    pltpu.async_copy(cart_hbm.at[idxn_v[0]], gn_v[0], gsem[0])
    pltpu.async_copy(cart_hbm.at[idxc_v[0]], gc_v[0], gsem[0])

    def quad(i4, carry):
        c0 = i4 * 4
        step(c0, 0, 1, 2, 0, 1, True)
        step(c0 + 1, 1, 2, 3, 1, 0, True)
        step(c0 + 2, 2, 3, 0, 0, 1, False)
        step(c0 + 3, 3, 0, 1, 1, 0, False)
        return carry

    lax.fori_loop(0, NCHUNK // 4, quad, 0)
    wait_streams(1)
    wait_gathers(0, 0)
    wait_store(194, 0)
    wait_store(195, 1)


def _sc_setup(cartp, idxn, idxc, shifts_t):
    mesh = plsc.VectorSubcoreMesh(core_axis_name="c", subcore_axis_name="s")
    f = pl.kernel(
        _sc_setup_body,
        out_type=jax.ShapeDtypeStruct((4, EP), _f32),
        mesh=mesh,
        compiler_params=pltpu.CompilerParams(needs_layout_passes=False, use_tc_tiling_on_sc=False),
        scratch_types=(
            [pltpu.VMEM((C,), jnp.int32)] * 8
            + [pltpu.VMEM((4, C), _f32)] * 4
            + [pltpu.VMEM((C, 8), _f32)] * 4
            + [pltpu.VMEM((4, C), _f32)] * 2
            + [pltpu.SemaphoreType.DMA] * 8
        ),
    )
    return f(cartp, idxn, idxc, shifts_t)


def _sc_edge_body(with_mpc, p_hbm, ec_hbm, idxn_hbm, idxc_hbm, out_hbm,
                  idxn_v0, idxn_v1, idxn_v2, idxn_v3,
                  idxc_v0, idxc_v1, idxc_v2, idxc_v3,
                  ec_v0, ec_v1, ec_v2, ec_v3,
                  rows_v0, rows_v1, out_v0, out_v1, zb_v, acc_sh,
                  asem0, asem1, asem2, asem3, gsem0, gsem1, ssem0, ssem1):
    cid = lax.axis_index("c")
    sid = lax.axis_index("s")
    idxn_v = (idxn_v0, idxn_v1, idxn_v2, idxn_v3)
    idxc_v = (idxc_v0, idxc_v1, idxc_v2, idxc_v3)
    ec_v = (ec_v0, ec_v1, ec_v2, ec_v3)
    rows_v = (rows_v0, rows_v1)
    out_v = (out_v0, out_v1)
    asem = (asem0, asem1, asem2, asem3)
    gsem = (gsem0, gsem1)
    ssem = (ssem0, ssem1)

    def zrow(r, carry):
        zb_v[r, pl.ds(0, 16)] = jnp.zeros((16,), _f32)
        zb_v[r, pl.ds(16, 16)] = jnp.zeros((16,), _f32)
        return carry

    lax.fori_loop(0, ZB, zrow, 0)

    def zcp(i, carry):
        pltpu.sync_copy(zb_v, acc_sh.at[pl.ds(sid * TROWS + i * ZB, ZB)])
        return carry

    lax.fori_loop(0, NZCP, zcp, 0)
    plsc.subcore_barrier()

    base = (sid * NC + cid) * ET

    def issue_streams(c, q):
        e1 = base + c * C
        pltpu.async_copy(idxn_hbm.at[pl.ds(e1, C)], idxn_v[q], asem[q])
        pltpu.async_copy(idxc_hbm.at[pl.ds(e1, C)], idxc_v[q], asem[q])
        pltpu.async_copy(ec_hbm.at[:, pl.ds(e1, C)], ec_v[q], asem[q])

    def wait_streams(q):
        pltpu.make_async_copy(idxn_hbm.at[pl.ds(base, C)], idxn_v[q],
                              asem[q]).wait()
        pltpu.make_async_copy(idxc_hbm.at[pl.ds(base, C)], idxc_v[q],
                              asem[q]).wait()
        pltpu.make_async_copy(ec_hbm.at[:, pl.ds(base, C)], ec_v[q],
                              asem[q]).wait()

    def compute(b, q):
        def group2(g2, carry):
            for gg in range(2):
                rid = lax.iota(jnp.int32, 16) + (g2 * 2 + gg) * 16
                cut = plsc.load_gather(ec_v[q], [_full16(0), rid])
                cx = plsc.load_gather(ec_v[q], [_full16(1), rid])
                cy = plsc.load_gather(ec_v[q], [_full16(2), rid])
                cz = plsc.load_gather(ec_v[q], [_full16(3), rid])
                for k in range(NWAVE):
                    nck = plsc.load_gather(rows_v[b], [rid, _full16(k)])
                    plsc.store_scatter(out_v[b], [rid, _full16(k)], cut * nck)
                for k in range(NWAVE):
                    nc2 = plsc.load_gather(rows_v[b], [rid, _full16(8 + k)])
                    for j, cj in enumerate((cx, cy, cz)):
                        if with_mpc:
                            mpcv = plsc.load_gather(
                                rows_v[b], [rid, _full16(16 + j * 8 + k)])
                            val = cj * nc2 + mpcv
                        else:
                            val = cj * nc2
                        plsc.store_scatter(out_v[b],
                                           [rid, _full16(8 + j * 8 + k)], val)
            return carry

        lax.fori_loop(0, C // 32, group2, 0)

    def wait_scatter(b, q):
        pltpu.make_async_copy(out_v[b], acc_sh.at[idxc_v[q]], ssem[b]).wait()

    def wait_gather(b, q):
        pltpu.make_async_copy(p_hbm.at[idxn_v[q]], rows_v[b], gsem[b]).wait()

    def step(c, q, q1, q2, q3, b, nb, first):
        wait_streams(q1)
        pltpu.async_copy(p_hbm.at[idxn_v[q1]], rows_v[nb], gsem[nb])
        issue_streams(c + 2, q2)
        wait_gather(b, q)
        compute(b, q)
        if first:
            @pl.when(c >= 1)
            def _w():
                wait_scatter(nb, q3)
        else:
            wait_scatter(nb, q3)
        pltpu.async_copy(out_v[b], acc_sh.at[idxc_v[q]], ssem[b], add=True)

    pltpu.sync_copy(idxn_hbm.at[pl.ds(base, C)], idxn_v[0])
    pltpu.sync_copy(idxc_hbm.at[pl.ds(base, C)], idxc_v[0])
    pltpu.sync_copy(ec_hbm.at[:, pl.ds(base, C)], ec_v[0])
    issue_streams(1, 1)
    pltpu.async_copy(p_hbm.at[idxn_v[0]], rows_v[0], gsem[0])

    def quad(i4, carry):
        c0 = i4 * 4
        step(c0, 0, 1, 2, 3, 0, 1, True)
        step(c0 + 1, 1, 2, 3, 0, 1, 0, False)
        step(c0 + 2, 2, 3, 0, 1, 0, 1, False)
        step(c0 + 3, 3, 0, 1, 2, 1, 0, False)
        return carry

    lax.fori_loop(0, NCHUNK // 4, quad, 0)
    wait_streams(1)
    wait_gather(0, 0)
    wait_scatter(1, 3)
    plsc.subcore_barrier()
    r0 = sid * TROWS
    pltpu.sync_copy(acc_sh.at[pl.ds(r0, TROWS)],
                    out_hbm.at[cid, pl.ds(r0, TROWS)])


def _sc_edge_call(p, ect, idxn, idxc, pw, with_mpc):
    mesh = plsc.VectorSubcoreMesh(core_axis_name="c", subcore_axis_name="s")
    f = pl.kernel(
        functools.partial(_sc_edge_body, with_mpc),
        out_type=jax.ShapeDtypeStruct((NC, NP, 32), _f32),
        mesh=mesh,
        compiler_params=pltpu.CompilerParams(needs_layout_passes=False, use_tc_tiling_on_sc=False),
        scratch_types=(
            [pltpu.VMEM((C,), jnp.int32)] * 8
            + [pltpu.VMEM((4, C), _f32)] * 4
            + [pltpu.VMEM((C, pw), _f32)] * 2
            + [pltpu.VMEM((C, 32), _f32)] * 2
            + [pltpu.VMEM((ZB, 32), _f32)]
            + [pltpu.VMEM_SHARED((NP, 32), _f32)]
            + [pltpu.SemaphoreType.DMA] * 8
        ),
    )
    return f(p, ect, idxn, idxc)


def _sc_edge(p, ect, idxn, idxc):
    return _sc_edge_call(p, ect, idxn, idxc, 40, True)


def _sc_edge0(p0, ect, idxn, idxc):
    return _sc_edge_call(p0, ect, idxn, idxc, 16, False)


def _tc_prep_body(dv_ref, ec_ref):
    pid = pl.program_id(0)
    dv = dv_ref[...]
    dx = dv[0:1, :]
    dy = dv[1:2, :]
    dz = dv[2:3, :]
    r2 = dv[3:4, :]
    d = jnp.sqrt(r2)
    cut = jnp.power(0.5 * jnp.cos(d * (np.pi / CUTOFF)) + 0.5, 3)
    col = lax.broadcasted_iota(jnp.int32, (1, BE), 1) + pid * BE
    cut = jnp.where(col < N_EDGES, cut, 0.0)
    ec_ref[...] = jnp.concatenate([cut, cut * dx, cut * dy, cut * dz], axis=0)


def _tc_prep(dvt):
    return pl.pallas_call(
        _tc_prep_body,
        grid=(GE,),
        in_specs=[pl.BlockSpec((4, BE), lambda i: (0, i))],
        out_specs=pl.BlockSpec((4, BE), lambda i: (0, i)),
        out_shape=jax.ShapeDtypeStruct((4, EP), _f32),
    )(dvt)


def _tc_emb_body(sp_ref, we_ref, be_ref, icf_ref, p_ref):
    pid = pl.program_id(0)
    icf = _silu(jnp.dot(sp_ref[...], we_ref[...],
                        preferred_element_type=_f32) + be_ref[...])
    icf_ref[...] = icf
    rows = lax.broadcasted_iota(jnp.int32, (BN, 1), 0) + pid * BN
    mask = (rows < N_NODES).astype(_f32)
    p = jnp.concatenate([icf[:, 0:8], icf[:, 9:17]], axis=1)
    p_ref[...] = p * mask


def _tc_emb(speciesp, w_emb, b_emb):
    return pl.pallas_call(
        _tc_emb_body,
        grid=(GN,),
        in_specs=[
            pl.BlockSpec((BN, NSPEC), lambda i: (i, 0)),
            pl.BlockSpec((NSPEC, NCOEF), lambda i: (0, 0)),
            pl.BlockSpec((1, NCOEF), lambda i: (0, 0)),
        ],
        out_specs=[
            pl.BlockSpec((BN, NCOEF), lambda i: (i, 0)),
            pl.BlockSpec((BN, 16), lambda i: (i, 0)),
        ],
        out_shape=[
            jax.ShapeDtypeStruct((NP, NCOEF), _f32),
            jax.ShapeDtypeStruct((NP, 16), _f32),
        ],
    )(speciesp, w_emb, b_emb)


def _tc_node_body(acc0_ref, acc1_ref, icf_ref, dens_ref, mpc_ref,
                  wr_ref, br_ref, wm_ref, bm_ref,
                  dens_out, mpc_out, icf_out, p_out):
    pid = pl.program_id(0)
    acc = acc0_ref[0] + acc1_ref[0]
    mpd = acc[:, 0:8]
    ss = acc[:, 8:32]
    icf = icf_ref[...]
    mpc = mpc_ref[...] + ss
    radial = _silu(jnp.dot(mpd * icf[:, 8:9], wr_ref[...],
                           preferred_element_type=_f32) + br_ref[...])
    c = icf[:, 17:18]
    x = mpc[:, 0:8] * c
    y = mpc[:, 8:16] * c
    z = mpc[:, 16:24] * c
    r2 = x * x + y * y + z * z
    ang2 = ((x * y) ** 2 + (y * z) ** 2 + (3.0 * z * z - r2) ** 2
            + (x * z) ** 2 + (x * x - y * y) ** 2)
    dens = dens_ref[...] + jnp.concatenate(
        [radial, radial * r2, radial * ang2], axis=1)
    dens_out[...] = dens
    mpc_out[...] = mpc
    icf_new = _silu(jnp.dot(dens, wm_ref[...],
                            preferred_element_type=_f32) + bm_ref[...])
    icf_out[...] = icf_new
    rows = lax.broadcasted_iota(jnp.int32, (BN, 1), 0) + pid * BN
    mask = (rows < N_NODES).astype(_f32)
    p = jnp.concatenate([icf_new[:, 0:8], icf_new[:, 9:17], mpc], axis=1)
    p_out[...] = p * mask


def _tc_node(acc, icf, dens, mpc, w_rad, b_rad, w_msg_i, b_msg_i):
    return pl.pallas_call(
        _tc_node_body,
        grid=(GN,),
        in_specs=[
            pl.BlockSpec((1, BN, 32), lambda i: (0, i, 0)),
            pl.BlockSpec((1, BN, 32), lambda i: (1, i, 0)),
            pl.BlockSpec((BN, NCOEF), lambda i: (i, 0)),
            pl.BlockSpec((BN, NORB), lambda i: (i, 0)),
            pl.BlockSpec((BN, NORB), lambda i: (i, 0)),
            pl.BlockSpec((NWAVE, NWAVE), lambda i: (0, 0)),
            pl.BlockSpec((1, NWAVE), lambda i: (0, 0)),
            pl.BlockSpec((NORB, NCOEF), lambda i: (0, 0)),
            pl.BlockSpec((1, NCOEF), lambda i: (0, 0)),
        ],
        out_specs=[
            pl.BlockSpec((BN, NORB), lambda i: (i, 0)),
            pl.BlockSpec((BN, NORB), lambda i: (i, 0)),
            pl.BlockSpec((BN, NCOEF), lambda i: (i, 0)),
            pl.BlockSpec((BN, 40), lambda i: (i, 0)),
        ],
        out_shape=[
            jax.ShapeDtypeStruct((NP, NORB), _f32),
            jax.ShapeDtypeStruct((NP, NORB), _f32),
            jax.ShapeDtypeStruct((NP, NCOEF), _f32),
            jax.ShapeDtypeStruct((NP, 40), _f32),
        ],
    )(acc, acc, icf, dens, mpc, w_rad, b_rad, w_msg_i, b_msg_i)


def _tc_node_last_body(acc0_ref, acc1_ref, icf_ref, dens_ref, mpc_ref,
                       wr_ref, br_ref, wo_ref, out_ref):
    pid = pl.program_id(0)
    acc = acc0_ref[0] + acc1_ref[0]
    mpd = acc[:, 0:8]
    ss = acc[:, 8:32]
    icf = icf_ref[...]
    mpc = mpc_ref[...] + ss
    radial = _silu(jnp.dot(mpd * icf[:, 8:9], wr_ref[...],
                           preferred_element_type=_f32) + br_ref[...])
    c = icf[:, 17:18]
    x = mpc[:, 0:8] * c
    y = mpc[:, 8:16] * c
    z = mpc[:, 16:24] * c
    r2 = x * x + y * y + z * z
    ang2 = ((x * y) ** 2 + (y * z) ** 2 + (3.0 * z * z - r2) ** 2
            + (x * z) ** 2 + (x * x - y * y) ** 2)
    dens = dens_ref[...] + jnp.concatenate(
        [radial, radial * r2, radial * ang2], axis=1)
    rows = lax.broadcasted_iota(jnp.int32, (BN, 1), 0) + pid * BN
    mask = (rows < N_NODES).astype(_f32)
    part = jnp.sum(jnp.dot(dens * mask, wo_ref[...],
                           preferred_element_type=_f32))

    @pl.when(pid == 0)
    def _init():
        out_ref[...] = jnp.zeros((1, 1), _f32)

    out_ref[...] = out_ref[...] + part


def _tc_node_last(acc, icf, dens, mpc, w_rad, b_rad, w_out):
    return pl.pallas_call(
        _tc_node_last_body,
        grid=(GN,),
        in_specs=[
            pl.BlockSpec((1, BN, 32), lambda i: (0, i, 0)),
            pl.BlockSpec((1, BN, 32), lambda i: (1, i, 0)),
            pl.BlockSpec((BN, NCOEF), lambda i: (i, 0)),
            pl.BlockSpec((BN, NORB), lambda i: (i, 0)),
            pl.BlockSpec((BN, NORB), lambda i: (i, 0)),
            pl.BlockSpec((NWAVE, NWAVE), lambda i: (0, 0)),
            pl.BlockSpec((1, NWAVE), lambda i: (0, 0)),
            pl.BlockSpec((NORB, 1), lambda i: (0, 0)),
        ],
        out_specs=pl.BlockSpec((1, 1), lambda i: (0, 0)),
        out_shape=jax.ShapeDtypeStruct((1, 1), _f32),
    )(acc, acc, icf, dens, mpc, w_rad, b_rad, w_out)


def kernel(cart, neighlist, shifts, species, W_emb, b_emb, W_rad, b_rad,
           W_msg, b_msg, W_out, b_out):
    idx_c = neighlist[0].astype(jnp.int32)
    idx_n = neighlist[1].astype(jnp.int32)
    idx_c = jnp.pad(idx_c, (0, EPP - N_EDGES), constant_values=N_NODES)
    idx_n = jnp.pad(idx_n, (0, EPP - N_EDGES), constant_values=N_NODES)
    shifts_t = jnp.pad(shifts.T.astype(_f32), ((0, 1), (0, EPP - N_EDGES)))
    cartp = jnp.pad(cart.astype(_f32), ((0, NP - N_NODES), (0, 5)))
    speciesp = jnp.pad(species.astype(_f32), ((0, NP - N_NODES), (0, 0)))

    dvt = _sc_setup(cartp, idx_n, idx_c, shifts_t)
    ect = jnp.pad(_tc_prep(dvt), ((0, 0), (0, EPP - EP)))
    icf, p = _tc_emb(speciesp, W_emb, b_emb.reshape(1, NCOEF))

    dens = jnp.zeros((NP, NORB), _f32)
    mpc = jnp.zeros((NP, NORB), _f32)
    b_rad2 = b_rad.reshape(1, NWAVE)
    for i in range(ITER_LOOP):
        acc = _sc_edge0(p, ect, idx_n, idx_c) if i == 0 else \
            _sc_edge(p, ect, idx_n, idx_c)
        dens, mpc, icf, p = _tc_node(acc, icf, dens, mpc, W_rad, b_rad2,
                                     W_msg[i], b_msg[i].reshape(1, NCOEF))
    acc = _sc_edge(p, ect, idx_n, idx_c)
    out = _tc_node_last(acc, icf, dens, mpc, W_rad, b_rad2, W_out)
    return out[0, 0] + N_NODES * b_out[0]
